# baseline (speedup 1.0000x reference)
import functools

import jax
import jax.numpy as jnp
from jax import lax
from jax.experimental import pallas as pl
from jax.experimental.pallas import tpu as pltpu
from jax.experimental.pallas import tpu_sc as plsc

N_NODES = 10000
D = 128
E = 320000
G = 64

NC = 2
NS = 16
NW = NC * NS
E_PER_W = E // NW
CH = 80
NCHUNK = E_PER_W // CH
NBUF = 4
NSTAGE = 5
NL = NCHUNK // NSTAGE
NITER = (NL + NBUF - 1) // NBUF
ROWS_PER_TILE = N_NODES // NS
NFULL = ROWS_PER_TILE // CH
REM = ROWS_PER_TILE - NFULL * CH


def _sc_agg_body(h_hbm, src_hbm, dst_hbm, zero_hbm, out_hbm,
                 src_v, dst_v, rows0_v, rows1_v, rows2_v, rows3_v, agg_sh,
                 sem0, sem1, sem2, sem3):
    c = lax.axis_index("c")
    s = lax.axis_index("s")
    wid = s * NC + c
    stripe = s * ROWS_PER_TILE
    pltpu.sync_copy(zero_hbm, rows0_v)
    for z in range(NFULL):
        pltpu.sync_copy(rows0_v, agg_sh.at[pl.ds(stripe + z * CH, CH)])
    pltpu.sync_copy(rows0_v.at[pl.ds(0, REM)],
                    agg_sh.at[pl.ds(stripe + NFULL * CH, REM)])
    plsc.subcore_barrier()
    bufs = (rows0_v, rows1_v, rows2_v, rows3_v)
    sems = (sem0, sem1, sem2, sem3)
    for stage in range(NSTAGE):
        pltpu.sync_copy(src_hbm.at[wid].at[pl.ds(stage * NL, NL)], src_v)
        pltpu.sync_copy(dst_hbm.at[wid].at[pl.ds(stage * NL, NL)], dst_v)
        for p in range(NBUF - 1):
            pltpu.async_copy(h_hbm.at[src_v.at[p]], bufs[p], sems[p])

        def block(i, carry):
            g0 = i * NBUF
            for b in range(NBUF):
                g = g0 + b
                buf, sem = bufs[b], sems[b]
                pre = g + NBUF - 1
                pb = (b + NBUF - 1) % NBUF

                @pl.when(pre < NL)
                def _():
                    pltpu.async_copy(h_hbm.at[src_v.at[pre]], bufs[pb], sems[pb])

                @pl.when(g < NL)
                def _():
                    pltpu.make_async_copy(h_hbm.at[src_v.at[g]], buf, sem).wait()
                    pltpu.sync_copy(buf, agg_sh.at[dst_v.at[g]], add=True)
            return carry

        lax.fori_loop(0, NITER, block, 0)
    plsc.subcore_barrier()
    for z in range(NFULL):
        base = stripe + z * CH
        pltpu.sync_copy(agg_sh.at[pl.ds(base, CH)], rows0_v)
        pltpu.sync_copy(rows0_v, out_hbm.at[c].at[pl.ds(base, CH)])
    base = stripe + NFULL * CH
    pltpu.sync_copy(agg_sh.at[pl.ds(base, REM)], rows1_v.at[pl.ds(0, REM)])
    pltpu.sync_copy(rows1_v.at[pl.ds(0, REM)], out_hbm.at[c].at[pl.ds(base, REM)])


def _sc_aggregate(h, src3, dst3, zeros):
    return pl.kernel(
        _sc_agg_body,
        out_type=jax.ShapeDtypeStruct((NC, N_NODES, D), jnp.float32),
        mesh=plsc.VectorSubcoreMesh(
            core_axis_name="c", subcore_axis_name="s",
            num_cores=NC, num_subcores=NS),
        scratch_types=[
            pltpu.VMEM((NL, CH), jnp.int32),
            pltpu.VMEM((NL, CH), jnp.int32),
            pltpu.VMEM((CH, D), jnp.float32),
            pltpu.VMEM((CH, D), jnp.float32),
            pltpu.VMEM((CH, D), jnp.float32),
            pltpu.VMEM((CH, D), jnp.float32),
            pltpu.VMEM_SHARED((N_NODES, D), jnp.float32),
            pltpu.SemaphoreType.DMA,
            pltpu.SemaphoreType.DMA,
            pltpu.SemaphoreType.DMA,
            pltpu.SemaphoreType.DMA,
        ],
        compiler_params=pltpu.CompilerParams(use_tc_tiling_on_sc=False),
    )(h, src3, dst3, zeros)


def _tc_update_body(s_ref, x_ref, a0_ref, a1_ref, w_ref, b_ref, o_ref):
    t = s_ref[0] * x_ref[...] + a0_ref[...] + a1_ref[...]
    h = jnp.dot(t, w_ref[...], preferred_element_type=jnp.float32) + b_ref[...]
    o_ref[...] = jnp.maximum(h, 0.0)


def _tc_update(scale, h, a0, a1, w, b):
    return pl.pallas_call(
        _tc_update_body,
        out_shape=jax.ShapeDtypeStruct((N_NODES, D), jnp.float32),
        in_specs=[pl.BlockSpec(memory_space=pltpu.SMEM)]
        + [pl.BlockSpec(memory_space=pltpu.VMEM)] * 5,
        out_specs=pl.BlockSpec(memory_space=pltpu.VMEM),
    )(scale, h, a0, a1, w, b)


def _tc_final_body(s_ref, h_ref, a0_ref, a1_ref, w2_ref, b2_ref,
                   bat_ref, w3_ref, b3_ref, o_ref):
    t = s_ref[0] * h_ref[...] + a0_ref[...] + a1_ref[...]
    h2 = jnp.maximum(
        jnp.dot(t, w2_ref[...], preferred_element_type=jnp.float32) + b2_ref[...],
        0.0)
    gid = lax.broadcasted_iota(jnp.int32, (N_NODES, G), 1)
    onehot = (bat_ref[...] == gid).astype(jnp.float32)
    sums = lax.dot_general(onehot, h2, (((0,), (0,)), ((), ())),
                           preferred_element_type=jnp.float32)
    counts = jnp.sum(onehot, axis=0)
    pooled = sums / jnp.maximum(counts, 1.0)[:, None]
    o_ref[...] = jnp.dot(pooled, w3_ref[...],
                         preferred_element_type=jnp.float32) + b3_ref[...]


def _tc_final(scale, h, a0, a1, w2, b2, bat, w3, b3):
    return pl.pallas_call(
        _tc_final_body,
        out_shape=jax.ShapeDtypeStruct((G, D), jnp.float32),
        in_specs=[pl.BlockSpec(memory_space=pltpu.SMEM)]
        + [pl.BlockSpec(memory_space=pltpu.VMEM)] * 8,
        out_specs=pl.BlockSpec(memory_space=pltpu.VMEM),
    )(scale, h, a0, a1, w2, b2, bat, w3, b3)


def kernel(x, edge_index, batch, eps1, W1, b1, eps2, W2, b2, W3, b3):
    src3 = edge_index[0].astype(jnp.int32).reshape(NW, NCHUNK, CH)
    dst3 = edge_index[1].astype(jnp.int32).reshape(NW, NCHUNK, CH)
    zeros = jnp.zeros((CH, D), jnp.float32)
    s1 = (1.0 + eps1).reshape(1)
    s2 = (1.0 + eps2).reshape(1)
    b1r = b1.reshape(1, D)
    b2r = b2.reshape(1, D)
    b3r = b3.reshape(1, D)
    bat = batch.astype(jnp.int32).reshape(N_NODES, 1)

    agg1 = _sc_aggregate(x, src3, dst3, zeros)
    h1 = _tc_update(s1, x, agg1[0], agg1[1], W1, b1r)
    agg2 = _sc_aggregate(h1, src3, dst3, zeros)
    out = _tc_final(s2, h1, agg2[0], agg2[1], W2, b2r, bat, W3, b3r)
    return out


# speedup vs baseline: 1.0402x; 1.0402x over previous
import functools

import jax
import jax.numpy as jnp
from jax import lax
from jax.experimental import pallas as pl
from jax.experimental.pallas import tpu as pltpu
from jax.experimental.pallas import tpu_sc as plsc

N_NODES = 10000
D = 128
E = 320000
G = 64

NC = 2
NS = 16
NW = NC * NS
E_PER_W = E // NW
CH = 100
NCHUNK = E_PER_W // CH
NBUF = 3
NSTAGE = 4
NL = NCHUNK // NSTAGE
NITER = (NL + NBUF - 1) // NBUF
ROWS_PER_TILE = N_NODES // NS
NFULL = ROWS_PER_TILE // CH
REM = ROWS_PER_TILE - NFULL * CH


def _sc_agg_body(h_hbm, src_hbm, dst_hbm, zero_hbm, out_hbm,
                 src_va, dst_va, src_vb, dst_vb,
                 rows0_v, rows1_v, rows2_v, agg_sh,
                 sem0, sem1, sem2, semi):
    c = lax.axis_index("c")
    s = lax.axis_index("s")
    wid = s * NC + c
    bufs = (rows0_v, rows1_v, rows2_v)
    sems = (sem0, sem1, sem2)
    idxs = ((src_va, dst_va), (src_vb, dst_vb))
    pltpu.sync_copy(src_hbm.at[wid].at[pl.ds(0, NL)], src_va)
    pltpu.sync_copy(dst_hbm.at[wid].at[pl.ds(0, NL)], dst_va)
    for p in range(NBUF - 1):
        pltpu.async_copy(h_hbm.at[src_va.at[p]], bufs[p], sems[p])
    stripe = s * ROWS_PER_TILE
    zbuf = bufs[NBUF - 1]
    pltpu.sync_copy(zero_hbm, zbuf)
    for z in range(NFULL):
        pltpu.sync_copy(zbuf, agg_sh.at[pl.ds(stripe + z * CH, CH)])
    pltpu.sync_copy(zbuf.at[pl.ds(0, REM)],
                    agg_sh.at[pl.ds(stripe + NFULL * CH, REM)])
    plsc.subcore_barrier()
    for stage in range(NSTAGE):
        src_v, dst_v = idxs[stage % 2]
        if stage + 1 < NSTAGE:
            nsrc, ndst = idxs[(stage + 1) % 2]
            pltpu.async_copy(src_hbm.at[wid].at[pl.ds((stage + 1) * NL, NL)],
                             nsrc, semi)
            pltpu.async_copy(dst_hbm.at[wid].at[pl.ds((stage + 1) * NL, NL)],
                             ndst, semi)

        def block(i, carry):
            g0 = i * NBUF
            for b in range(NBUF):
                g = g0 + b
                buf, sem = bufs[b], sems[b]
                pre = g + NBUF - 1
                pb = (b + NBUF - 1) % NBUF

                @pl.when(pre < NL)
                def _():
                    pltpu.async_copy(h_hbm.at[src_v.at[pre]], bufs[pb], sems[pb])

                @pl.when(g < NL)
                def _():
                    pltpu.make_async_copy(h_hbm.at[src_v.at[g]], buf, sem).wait()
                    pltpu.sync_copy(buf, agg_sh.at[dst_v.at[g]], add=True)
            return carry

        lax.fori_loop(0, NITER, block, 0)
        if stage + 1 < NSTAGE:
            nsrc, ndst = idxs[(stage + 1) % 2]
            pltpu.make_async_copy(
                src_hbm.at[wid].at[pl.ds((stage + 1) * NL, NL)], nsrc, semi).wait()
            pltpu.make_async_copy(
                dst_hbm.at[wid].at[pl.ds((stage + 1) * NL, NL)], ndst, semi).wait()
            for p in range(NBUF - 1):
                pltpu.async_copy(h_hbm.at[nsrc.at[p]], bufs[p], sems[p])
    plsc.subcore_barrier()
    for z in range(NFULL):
        base = stripe + z * CH
        pltpu.sync_copy(agg_sh.at[pl.ds(base, CH)], rows0_v)
        pltpu.sync_copy(rows0_v, out_hbm.at[c].at[pl.ds(base, CH)])
    base = stripe + NFULL * CH
    pltpu.sync_copy(agg_sh.at[pl.ds(base, REM)], rows1_v.at[pl.ds(0, REM)])
    pltpu.sync_copy(rows1_v.at[pl.ds(0, REM)], out_hbm.at[c].at[pl.ds(base, REM)])


def _sc_aggregate(h, src3, dst3, zeros):
    return pl.kernel(
        _sc_agg_body,
        out_type=jax.ShapeDtypeStruct((NC, N_NODES, D), jnp.float32),
        mesh=plsc.VectorSubcoreMesh(
            core_axis_name="c", subcore_axis_name="s",
            num_cores=NC, num_subcores=NS),
        scratch_types=[
            pltpu.VMEM((NL, CH), jnp.int32),
            pltpu.VMEM((NL, CH), jnp.int32),
            pltpu.VMEM((NL, CH), jnp.int32),
            pltpu.VMEM((NL, CH), jnp.int32),
            pltpu.VMEM((CH, D), jnp.float32),
            pltpu.VMEM((CH, D), jnp.float32),
            pltpu.VMEM((CH, D), jnp.float32),
            pltpu.VMEM_SHARED((N_NODES, D), jnp.float32),
            pltpu.SemaphoreType.DMA,
            pltpu.SemaphoreType.DMA,
            pltpu.SemaphoreType.DMA,
            pltpu.SemaphoreType.DMA,
        ],
        compiler_params=pltpu.CompilerParams(use_tc_tiling_on_sc=False),
    )(h, src3, dst3, zeros)


def _tc_update_body(s_ref, x_ref, a0_ref, a1_ref, w_ref, b_ref, o_ref):
    t = s_ref[0] * x_ref[...] + a0_ref[...] + a1_ref[...]
    h = jnp.dot(t, w_ref[...], preferred_element_type=jnp.float32) + b_ref[...]
    o_ref[...] = jnp.maximum(h, 0.0)


def _tc_update(scale, h, a0, a1, w, b):
    return pl.pallas_call(
        _tc_update_body,
        out_shape=jax.ShapeDtypeStruct((N_NODES, D), jnp.float32),
        in_specs=[pl.BlockSpec(memory_space=pltpu.SMEM)]
        + [pl.BlockSpec(memory_space=pltpu.VMEM)] * 5,
        out_specs=pl.BlockSpec(memory_space=pltpu.VMEM),
    )(scale, h, a0, a1, w, b)


def _tc_final_body(s_ref, h_ref, a0_ref, a1_ref, w2_ref, b2_ref,
                   bat_ref, w3_ref, b3_ref, o_ref):
    t = s_ref[0] * h_ref[...] + a0_ref[...] + a1_ref[...]
    h2 = jnp.maximum(
        jnp.dot(t, w2_ref[...], preferred_element_type=jnp.float32) + b2_ref[...],
        0.0)
    gid = lax.broadcasted_iota(jnp.int32, (N_NODES, G), 1)
    onehot = (bat_ref[...] == gid).astype(jnp.float32)
    sums = lax.dot_general(onehot, h2, (((0,), (0,)), ((), ())),
                           preferred_element_type=jnp.float32)
    counts = jnp.sum(onehot, axis=0)
    pooled = sums / jnp.maximum(counts, 1.0)[:, None]
    o_ref[...] = jnp.dot(pooled, w3_ref[...],
                         preferred_element_type=jnp.float32) + b3_ref[...]


def _tc_final(scale, h, a0, a1, w2, b2, bat, w3, b3):
    return pl.pallas_call(
        _tc_final_body,
        out_shape=jax.ShapeDtypeStruct((G, D), jnp.float32),
        in_specs=[pl.BlockSpec(memory_space=pltpu.SMEM)]
        + [pl.BlockSpec(memory_space=pltpu.VMEM)] * 8,
        out_specs=pl.BlockSpec(memory_space=pltpu.VMEM),
    )(scale, h, a0, a1, w2, b2, bat, w3, b3)


def kernel(x, edge_index, batch, eps1, W1, b1, eps2, W2, b2, W3, b3):
    src3 = edge_index[0].astype(jnp.int32).reshape(NW, NCHUNK, CH)
    dst3 = edge_index[1].astype(jnp.int32).reshape(NW, NCHUNK, CH)
    zeros = jnp.zeros((CH, D), jnp.float32)
    s1 = (1.0 + eps1).reshape(1)
    s2 = (1.0 + eps2).reshape(1)
    b1r = b1.reshape(1, D)
    b2r = b2.reshape(1, D)
    b3r = b3.reshape(1, D)
    bat = batch.astype(jnp.int32).reshape(N_NODES, 1)

    agg1 = _sc_aggregate(x, src3, dst3, zeros)
    h1 = _tc_update(s1, x, agg1[0], agg1[1], W1, b1r)
    agg2 = _sc_aggregate(h1, src3, dst3, zeros)
    out = _tc_final(s2, h1, agg2[0], agg2[1], W2, b2r, bat, W3, b3r)
    return out


# speedup vs baseline: 1.1589x; 1.1141x over previous
import jax
import jax.numpy as jnp
from jax import lax
from jax.experimental import pallas as pl
from jax.experimental.pallas import tpu as pltpu
from jax.experimental.pallas import tpu_sc as plsc

N_NODES = 10000
D = 128
E = 320000
G = 64

NC = 2
NS = 16
NW = NC * NS
E_PER_W = E // NW
CH = 80
NCHUNK = E_PER_W // CH
NBUF = 3
NSTAGE = 5
NL = NCHUNK // NSTAGE
EL = NL * CH
NITER = (NL + NBUF - 1) // NBUF
ROWS_PER_TILE = N_NODES // NS
NFULL = ROWS_PER_TILE // CH
REM = ROWS_PER_TILE - NFULL * CH


def _sc_agg_body(h_hbm, ei_hbm, zero_hbm, out_hbm,
                 src_va, dst_va, src_vb, dst_vb,
                 rows0_v, rows1_v, rows2_v, agg_sh,
                 sem0, sem1, sem2, semi):
    c = lax.axis_index("c")
    s = lax.axis_index("s")
    wid = s * NC + c
    ebase = wid * E_PER_W
    bufs = (rows0_v, rows1_v, rows2_v)
    sems = (sem0, sem1, sem2)
    idxs = ((src_va, dst_va), (src_vb, dst_vb))
    pltpu.sync_copy(ei_hbm.at[0].at[pl.ds(ebase, EL)], src_va)
    pltpu.sync_copy(ei_hbm.at[1].at[pl.ds(ebase, EL)], dst_va)
    for p in range(NBUF - 1):
        pltpu.async_copy(h_hbm.at[src_va.at[pl.ds(p * CH, CH)]],
                         bufs[p], sems[p])
    stripe = s * ROWS_PER_TILE
    zbuf = bufs[NBUF - 1]
    pltpu.sync_copy(zero_hbm, zbuf)
    for z in range(NFULL):
        pltpu.sync_copy(zbuf, agg_sh.at[pl.ds(stripe + z * CH, CH)])
    pltpu.sync_copy(zbuf.at[pl.ds(0, REM)],
                    agg_sh.at[pl.ds(stripe + NFULL * CH, REM)])
    plsc.subcore_barrier()
    for stage in range(NSTAGE):
        src_v, dst_v = idxs[stage % 2]
        if stage + 1 < NSTAGE:
            nsrc, ndst = idxs[(stage + 1) % 2]
            off = ebase + (stage + 1) * EL
            pltpu.async_copy(ei_hbm.at[0].at[pl.ds(off, EL)], nsrc, semi)
            pltpu.async_copy(ei_hbm.at[1].at[pl.ds(off, EL)], ndst, semi)

        def block(i, carry):
            g0 = i * NBUF
            for b in range(NBUF):
                g = g0 + b
                buf, sem = bufs[b], sems[b]
                pre = g + NBUF - 1
                pb = (b + NBUF - 1) % NBUF

                @pl.when(pre < NL)
                def _():
                    pltpu.async_copy(
                        h_hbm.at[src_v.at[pl.ds(pre * CH, CH)]],
                        bufs[pb], sems[pb])

                @pl.when(g < NL)
                def _():
                    pltpu.make_async_copy(
                        h_hbm.at[src_v.at[pl.ds(g * CH, CH)]], buf, sem).wait()
                    pltpu.sync_copy(
                        buf, agg_sh.at[dst_v.at[pl.ds(g * CH, CH)]], add=True)
            return carry

        lax.fori_loop(0, NITER, block, 0)
        if stage + 1 < NSTAGE:
            nsrc, ndst = idxs[(stage + 1) % 2]
            off = ebase + (stage + 1) * EL
            pltpu.make_async_copy(ei_hbm.at[0].at[pl.ds(off, EL)],
                                  nsrc, semi).wait()
            pltpu.make_async_copy(ei_hbm.at[1].at[pl.ds(off, EL)],
                                  ndst, semi).wait()
            for p in range(NBUF - 1):
                pltpu.async_copy(h_hbm.at[nsrc.at[pl.ds(p * CH, CH)]],
                                 bufs[p], sems[p])
    plsc.subcore_barrier()
    for z in range(NFULL):
        base = stripe + z * CH
        pltpu.sync_copy(agg_sh.at[pl.ds(base, CH)], rows0_v)
        pltpu.sync_copy(rows0_v, out_hbm.at[c].at[pl.ds(base, CH)])
    base = stripe + NFULL * CH
    pltpu.sync_copy(agg_sh.at[pl.ds(base, REM)], rows1_v.at[pl.ds(0, REM)])
    pltpu.sync_copy(rows1_v.at[pl.ds(0, REM)], out_hbm.at[c].at[pl.ds(base, REM)])


def _sc_aggregate(h, edge_index, zeros):
    return pl.kernel(
        _sc_agg_body,
        out_type=jax.ShapeDtypeStruct((NC, N_NODES, D), jnp.float32),
        mesh=plsc.VectorSubcoreMesh(
            core_axis_name="c", subcore_axis_name="s",
            num_cores=NC, num_subcores=NS),
        scratch_types=[
            pltpu.VMEM((EL,), jnp.int32),
            pltpu.VMEM((EL,), jnp.int32),
            pltpu.VMEM((EL,), jnp.int32),
            pltpu.VMEM((EL,), jnp.int32),
            pltpu.VMEM((CH, D), jnp.float32),
            pltpu.VMEM((CH, D), jnp.float32),
            pltpu.VMEM((CH, D), jnp.float32),
            pltpu.VMEM_SHARED((N_NODES, D), jnp.float32),
            pltpu.SemaphoreType.DMA,
            pltpu.SemaphoreType.DMA,
            pltpu.SemaphoreType.DMA,
            pltpu.SemaphoreType.DMA,
        ],
        compiler_params=pltpu.CompilerParams(use_tc_tiling_on_sc=False),
    )(h, edge_index, zeros)


def _tc_update_body(e_ref, x_ref, a_ref, w_ref, b_ref, o_ref):
    t = (1.0 + e_ref[0]) * x_ref[...] + a_ref[0] + a_ref[1]
    h = jnp.dot(t, w_ref[...], preferred_element_type=jnp.float32) + b_ref[...]
    o_ref[...] = jnp.maximum(h, 0.0)


def _tc_update(eps, h, agg, w, b):
    return pl.pallas_call(
        _tc_update_body,
        out_shape=jax.ShapeDtypeStruct((N_NODES, D), jnp.float32),
        in_specs=[pl.BlockSpec(memory_space=pltpu.SMEM)]
        + [pl.BlockSpec(memory_space=pltpu.VMEM)] * 4,
        out_specs=pl.BlockSpec(memory_space=pltpu.VMEM),
    )(eps, h, agg, w, b)


def _tc_final_body(e_ref, h_ref, a_ref, w2_ref, b2_ref,
                   bat_ref, w3_ref, b3_ref, o_ref):
    t = (1.0 + e_ref[0]) * h_ref[...] + a_ref[0] + a_ref[1]
    h2 = jnp.maximum(
        jnp.dot(t, w2_ref[...], preferred_element_type=jnp.float32) + b2_ref[...],
        0.0)
    gid = lax.broadcasted_iota(jnp.int32, (G, N_NODES), 0)
    onehot = (bat_ref[...].reshape(1, N_NODES) == gid).astype(jnp.float32)
    sums = jnp.dot(onehot, h2, preferred_element_type=jnp.float32)
    counts = jnp.sum(onehot, axis=1)
    pooled = sums / jnp.maximum(counts, 1.0)[:, None]
    o_ref[...] = jnp.dot(pooled, w3_ref[...],
                         preferred_element_type=jnp.float32) + b3_ref[...]


def _tc_final(eps, h, agg, w2, b2, bat, w3, b3):
    return pl.pallas_call(
        _tc_final_body,
        out_shape=jax.ShapeDtypeStruct((G, D), jnp.float32),
        in_specs=[pl.BlockSpec(memory_space=pltpu.SMEM)]
        + [pl.BlockSpec(memory_space=pltpu.VMEM)] * 7,
        out_specs=pl.BlockSpec(memory_space=pltpu.VMEM),
    )(eps, h, agg, w2, b2, bat, w3, b3)


def kernel(x, edge_index, batch, eps1, W1, b1, eps2, W2, b2, W3, b3):
    ei = edge_index.astype(jnp.int32)
    zeros = jnp.zeros((CH, D), jnp.float32)
    e1 = eps1.reshape(1)
    e2 = eps2.reshape(1)
    b1r = b1.reshape(1, D)
    b2r = b2.reshape(1, D)
    b3r = b3.reshape(1, D)
    bat = batch.astype(jnp.int32)

    agg1 = _sc_aggregate(x, ei, zeros)
    h1 = _tc_update(e1, x, agg1, W1, b1r)
    agg2 = _sc_aggregate(h1, ei, zeros)
    out = _tc_final(e2, h1, agg2, W2, b2r, bat, W3, b3r)
    return out


# speedup vs baseline: 1.1674x; 1.0073x over previous
import jax
import jax.numpy as jnp
from jax import lax
from jax.experimental import pallas as pl
from jax.experimental.pallas import tpu as pltpu
from jax.experimental.pallas import tpu_sc as plsc

N_NODES = 10000
D = 128
E = 320000
G = 64

NC = 2
NS = 16
NW = NC * NS
E_PER_W = E // NW
CH = 80
NCHUNK = E_PER_W // CH
NBUF = 3
NSTAGE = 5
NL = NCHUNK // NSTAGE
EL = NL * CH
NITER = (NL + NBUF - 1) // NBUF
ROWS_PER_TILE = N_NODES // NS
NFULL = ROWS_PER_TILE // CH
REM = ROWS_PER_TILE - NFULL * CH


def _sc_agg_body(h_hbm, ei_hbm, zero_hbm, out_hbm,
                 src_va, dst_va, src_vb, dst_vb,
                 rows0_v, rows1_v, rows2_v, agg_sh,
                 sem0, sem1, sem2, semi):
    c = lax.axis_index("c")
    s = lax.axis_index("s")
    wid = s * NC + c
    ebase = wid * E_PER_W
    bufs = (rows0_v, rows1_v, rows2_v)
    sems = (sem0, sem1, sem2)
    idxs = ((src_va, dst_va), (src_vb, dst_vb))
    pltpu.sync_copy(ei_hbm.at[0].at[pl.ds(ebase, EL)], src_va)
    pltpu.sync_copy(ei_hbm.at[1].at[pl.ds(ebase, EL)], dst_va)
    for p in range(NBUF - 1):
        pltpu.async_copy(h_hbm.at[src_va.at[pl.ds(p * CH, CH)]],
                         bufs[p], sems[p])
    stripe = s * ROWS_PER_TILE
    zbuf = bufs[NBUF - 1]
    pltpu.sync_copy(zero_hbm, zbuf)
    for z in range(NFULL):
        pltpu.sync_copy(zbuf, agg_sh.at[pl.ds(stripe + z * CH, CH)])
    pltpu.sync_copy(zbuf.at[pl.ds(0, REM)],
                    agg_sh.at[pl.ds(stripe + NFULL * CH, REM)])
    plsc.subcore_barrier()
    for stage in range(NSTAGE):
        src_v, dst_v = idxs[stage % 2]
        if stage + 1 < NSTAGE:
            nsrc, ndst = idxs[(stage + 1) % 2]
            off = ebase + (stage + 1) * EL
            pltpu.async_copy(ei_hbm.at[0].at[pl.ds(off, EL)], nsrc, semi)
            pltpu.async_copy(ei_hbm.at[1].at[pl.ds(off, EL)], ndst, semi)

        def block(i, carry):
            g0 = i * NBUF
            for b in range(NBUF):
                g = g0 + b
                buf, sem = bufs[b], sems[b]
                pre = g + NBUF - 1
                pb = (b + NBUF - 1) % NBUF

                @pl.when(pre < NL)
                def _():
                    pltpu.async_copy(
                        h_hbm.at[src_v.at[pl.ds(pre * CH, CH)]],
                        bufs[pb], sems[pb])

                @pl.when(g < NL)
                def _():
                    pltpu.make_async_copy(
                        h_hbm.at[src_v.at[pl.ds(g * CH, CH)]], buf, sem).wait()
                    pltpu.sync_copy(
                        buf, agg_sh.at[dst_v.at[pl.ds(g * CH, CH)]], add=True)
            return carry

        lax.fori_loop(0, NITER, block, 0)
        if stage + 1 < NSTAGE:
            nsrc, ndst = idxs[(stage + 1) % 2]
            off = ebase + (stage + 1) * EL
            pltpu.make_async_copy(ei_hbm.at[0].at[pl.ds(off, EL)],
                                  nsrc, semi).wait()
            pltpu.make_async_copy(ei_hbm.at[1].at[pl.ds(off, EL)],
                                  ndst, semi).wait()
            for p in range(NBUF - 1):
                pltpu.async_copy(h_hbm.at[nsrc.at[pl.ds(p * CH, CH)]],
                                 bufs[p], sems[p])
    plsc.subcore_barrier()
    pltpu.sync_copy(agg_sh.at[pl.ds(stripe, ROWS_PER_TILE)],
                    out_hbm.at[c].at[pl.ds(stripe, ROWS_PER_TILE)])


def _sc_aggregate(h, edge_index, zeros):
    return pl.kernel(
        _sc_agg_body,
        out_type=jax.ShapeDtypeStruct((NC, N_NODES, D), jnp.float32),
        mesh=plsc.VectorSubcoreMesh(
            core_axis_name="c", subcore_axis_name="s",
            num_cores=NC, num_subcores=NS),
        scratch_types=[
            pltpu.VMEM((EL,), jnp.int32),
            pltpu.VMEM((EL,), jnp.int32),
            pltpu.VMEM((EL,), jnp.int32),
            pltpu.VMEM((EL,), jnp.int32),
            pltpu.VMEM((CH, D), jnp.float32),
            pltpu.VMEM((CH, D), jnp.float32),
            pltpu.VMEM((CH, D), jnp.float32),
            pltpu.VMEM_SHARED((N_NODES, D), jnp.float32),
            pltpu.SemaphoreType.DMA,
            pltpu.SemaphoreType.DMA,
            pltpu.SemaphoreType.DMA,
            pltpu.SemaphoreType.DMA,
        ],
        compiler_params=pltpu.CompilerParams(use_tc_tiling_on_sc=False),
    )(h, edge_index, zeros)


def _tc_update_body(e_ref, x_ref, a_ref, w_ref, b_ref, o_ref):
    t = (1.0 + e_ref[0]) * x_ref[...] + a_ref[0] + a_ref[1]
    h = jnp.dot(t, w_ref[...], preferred_element_type=jnp.float32) + b_ref[...]
    o_ref[...] = jnp.maximum(h, 0.0)


def _tc_update(eps, h, agg, w, b):
    return pl.pallas_call(
        _tc_update_body,
        out_shape=jax.ShapeDtypeStruct((N_NODES, D), jnp.float32),
        in_specs=[pl.BlockSpec(memory_space=pltpu.SMEM)]
        + [pl.BlockSpec(memory_space=pltpu.VMEM)] * 4,
        out_specs=pl.BlockSpec(memory_space=pltpu.VMEM),
    )(eps, h, agg, w, b)


def _tc_final_body(e_ref, h_ref, a_ref, w2_ref, b2_ref,
                   bat_ref, w3_ref, b3_ref, o_ref):
    t = (1.0 + e_ref[0]) * h_ref[...] + a_ref[0] + a_ref[1]
    h2 = jnp.maximum(
        jnp.dot(t, w2_ref[...], preferred_element_type=jnp.float32) + b2_ref[...],
        0.0)
    gid = lax.broadcasted_iota(jnp.int32, (G, N_NODES), 0)
    onehot = (bat_ref[...].reshape(1, N_NODES) == gid).astype(jnp.float32)
    sums = jnp.dot(onehot, h2, preferred_element_type=jnp.float32)
    counts = jnp.sum(onehot, axis=1)
    pooled = sums / jnp.maximum(counts, 1.0)[:, None]
    o_ref[...] = jnp.dot(pooled, w3_ref[...],
                         preferred_element_type=jnp.float32) + b3_ref[...]


def _tc_final(eps, h, agg, w2, b2, bat, w3, b3):
    return pl.pallas_call(
        _tc_final_body,
        out_shape=jax.ShapeDtypeStruct((G, D), jnp.float32),
        in_specs=[pl.BlockSpec(memory_space=pltpu.SMEM)]
        + [pl.BlockSpec(memory_space=pltpu.VMEM)] * 7,
        out_specs=pl.BlockSpec(memory_space=pltpu.VMEM),
    )(eps, h, agg, w2, b2, bat, w3, b3)


def kernel(x, edge_index, batch, eps1, W1, b1, eps2, W2, b2, W3, b3):
    ei = edge_index.astype(jnp.int32)
    zeros = jnp.zeros((CH, D), jnp.float32)
    e1 = eps1.reshape(1)
    e2 = eps2.reshape(1)
    b1r = b1.reshape(1, D)
    b2r = b2.reshape(1, D)
    b3r = b3.reshape(1, D)
    bat = batch.astype(jnp.int32)

    agg1 = _sc_aggregate(x, ei, zeros)
    h1 = _tc_update(e1, x, agg1, W1, b1r)
    agg2 = _sc_aggregate(h1, ei, zeros)
    out = _tc_final(e2, h1, agg2, W2, b2r, bat, W3, b3r)
    return out


# speedup vs baseline: 1.1783x; 1.0094x over previous
import jax
import jax.numpy as jnp
from jax import lax
from jax.experimental import pallas as pl
from jax.experimental.pallas import tpu as pltpu
from jax.experimental.pallas import tpu_sc as plsc

N_NODES = 10000
D = 128
E = 320000
G = 64

NC = 2
NS = 16
NW = NC * NS
E_PER_W = E // NW
CH = 80
NCHUNK = E_PER_W // CH
NBUF = 4
NSTAGE = 5
NL = NCHUNK // NSTAGE
EL = NL * CH
NITER = (NL + NBUF - 1) // NBUF
ROWS_PER_TILE = N_NODES // NS
NFULL = ROWS_PER_TILE // CH
REM = ROWS_PER_TILE - NFULL * CH


def _sc_agg_body(h_hbm, ei_hbm, zero_hbm, out_hbm,
                 src_va, dst_va, src_vb, dst_vb,
                 rows0_v, rows1_v, rows2_v, rows3_v, agg_sh,
                 sem0, sem1, sem2, sem3, semi):
    c = lax.axis_index("c")
    s = lax.axis_index("s")
    wid = s * NC + c
    ebase = wid * E_PER_W
    bufs = (rows0_v, rows1_v, rows2_v, rows3_v)
    sems = (sem0, sem1, sem2, sem3)
    idxs = ((src_va, dst_va), (src_vb, dst_vb))
    pltpu.sync_copy(ei_hbm.at[0].at[pl.ds(ebase, EL)], src_va)
    pltpu.sync_copy(ei_hbm.at[1].at[pl.ds(ebase, EL)], dst_va)
    for p in range(NBUF - 1):
        pltpu.async_copy(h_hbm.at[src_va.at[pl.ds(p * CH, CH)]],
                         bufs[p], sems[p])
    stripe = s * ROWS_PER_TILE
    zbuf = bufs[NBUF - 1]
    pltpu.sync_copy(zero_hbm, zbuf)
    for z in range(NFULL):
        pltpu.sync_copy(zbuf, agg_sh.at[pl.ds(stripe + z * CH, CH)])
    pltpu.sync_copy(zbuf.at[pl.ds(0, REM)],
                    agg_sh.at[pl.ds(stripe + NFULL * CH, REM)])
    plsc.subcore_barrier()
    for stage in range(NSTAGE):
        src_v, dst_v = idxs[stage % 2]
        if stage + 1 < NSTAGE:
            nsrc, ndst = idxs[(stage + 1) % 2]
            off = ebase + (stage + 1) * EL
            pltpu.async_copy(ei_hbm.at[0].at[pl.ds(off, EL)], nsrc, semi)
            pltpu.async_copy(ei_hbm.at[1].at[pl.ds(off, EL)], ndst, semi)

        def block(i, carry):
            g0 = i * NBUF
            for b in range(NBUF):
                g = g0 + b
                buf, sem = bufs[b], sems[b]
                pre = g + NBUF - 1
                pb = (b + NBUF - 1) % NBUF

                @pl.when(pre < NL)
                def _():
                    pltpu.async_copy(
                        h_hbm.at[src_v.at[pl.ds(pre * CH, CH)]],
                        bufs[pb], sems[pb])

                @pl.when(g < NL)
                def _():
                    pltpu.make_async_copy(
                        h_hbm.at[src_v.at[pl.ds(g * CH, CH)]], buf, sem).wait()
                    pltpu.sync_copy(
                        buf, agg_sh.at[dst_v.at[pl.ds(g * CH, CH)]], add=True)
            return carry

        lax.fori_loop(0, NITER, block, 0)
        if stage + 1 < NSTAGE:
            nsrc, ndst = idxs[(stage + 1) % 2]
            off = ebase + (stage + 1) * EL
            pltpu.make_async_copy(ei_hbm.at[0].at[pl.ds(off, EL)],
                                  nsrc, semi).wait()
            pltpu.make_async_copy(ei_hbm.at[1].at[pl.ds(off, EL)],
                                  ndst, semi).wait()
            for p in range(NBUF - 1):
                pltpu.async_copy(h_hbm.at[nsrc.at[pl.ds(p * CH, CH)]],
                                 bufs[p], sems[p])
    plsc.subcore_barrier()
    pltpu.sync_copy(agg_sh.at[pl.ds(stripe, ROWS_PER_TILE)],
                    out_hbm.at[c].at[pl.ds(stripe, ROWS_PER_TILE)])


def _sc_aggregate(h, edge_index, zeros):
    return pl.kernel(
        _sc_agg_body,
        out_type=jax.ShapeDtypeStruct((NC, N_NODES, D), jnp.float32),
        mesh=plsc.VectorSubcoreMesh(
            core_axis_name="c", subcore_axis_name="s",
            num_cores=NC, num_subcores=NS),
        scratch_types=[
            pltpu.VMEM((EL,), jnp.int32),
            pltpu.VMEM((EL,), jnp.int32),
            pltpu.VMEM((EL,), jnp.int32),
            pltpu.VMEM((EL,), jnp.int32),
            pltpu.VMEM((CH, D), jnp.float32),
            pltpu.VMEM((CH, D), jnp.float32),
            pltpu.VMEM((CH, D), jnp.float32),
            pltpu.VMEM((CH, D), jnp.float32),
            pltpu.VMEM_SHARED((N_NODES, D), jnp.float32),
            pltpu.SemaphoreType.DMA,
            pltpu.SemaphoreType.DMA,
            pltpu.SemaphoreType.DMA,
            pltpu.SemaphoreType.DMA,
            pltpu.SemaphoreType.DMA,
        ],
        compiler_params=pltpu.CompilerParams(use_tc_tiling_on_sc=False),
    )(h, edge_index, zeros)


def _tc_update_body(e_ref, x_ref, a_ref, w_ref, b_ref, o_ref):
    t = (1.0 + e_ref[0]) * x_ref[...] + a_ref[0] + a_ref[1]
    h = jnp.dot(t, w_ref[...], preferred_element_type=jnp.float32) + b_ref[...]
    o_ref[...] = jnp.maximum(h, 0.0)


def _tc_update(eps, h, agg, w, b):
    return pl.pallas_call(
        _tc_update_body,
        out_shape=jax.ShapeDtypeStruct((N_NODES, D), jnp.float32),
        in_specs=[pl.BlockSpec(memory_space=pltpu.SMEM)]
        + [pl.BlockSpec(memory_space=pltpu.VMEM)] * 4,
        out_specs=pl.BlockSpec(memory_space=pltpu.VMEM),
    )(eps, h, agg, w, b)


def _tc_final_body(e_ref, h_ref, a_ref, w2_ref, b2_ref,
                   bat_ref, w3_ref, b3_ref, o_ref):
    t = (1.0 + e_ref[0]) * h_ref[...] + a_ref[0] + a_ref[1]
    h2 = jnp.maximum(
        jnp.dot(t, w2_ref[...], preferred_element_type=jnp.float32) + b2_ref[...],
        0.0)
    gid = lax.broadcasted_iota(jnp.int32, (G, N_NODES), 0)
    onehot = (bat_ref[...].reshape(1, N_NODES) == gid).astype(jnp.float32)
    sums = jnp.dot(onehot, h2, preferred_element_type=jnp.float32)
    counts = jnp.sum(onehot, axis=1)
    pooled = sums / jnp.maximum(counts, 1.0)[:, None]
    o_ref[...] = jnp.dot(pooled, w3_ref[...],
                         preferred_element_type=jnp.float32) + b3_ref[...]


def _tc_final(eps, h, agg, w2, b2, bat, w3, b3):
    return pl.pallas_call(
        _tc_final_body,
        out_shape=jax.ShapeDtypeStruct((G, D), jnp.float32),
        in_specs=[pl.BlockSpec(memory_space=pltpu.SMEM)]
        + [pl.BlockSpec(memory_space=pltpu.VMEM)] * 7,
        out_specs=pl.BlockSpec(memory_space=pltpu.VMEM),
    )(eps, h, agg, w2, b2, bat, w3, b3)


def kernel(x, edge_index, batch, eps1, W1, b1, eps2, W2, b2, W3, b3):
    ei = edge_index.astype(jnp.int32)
    zeros = jnp.zeros((CH, D), jnp.float32)
    e1 = eps1.reshape(1)
    e2 = eps2.reshape(1)
    b1r = b1.reshape(1, D)
    b2r = b2.reshape(1, D)
    b3r = b3.reshape(1, D)
    bat = batch.astype(jnp.int32)

    agg1 = _sc_aggregate(x, ei, zeros)
    h1 = _tc_update(e1, x, agg1, W1, b1r)
    agg2 = _sc_aggregate(h1, ei, zeros)
    out = _tc_final(e2, h1, agg2, W2, b2r, bat, W3, b3r)
    return out
